# baseline (speedup 1.0000x reference)
import functools

import jax
import jax.numpy as jnp
from jax import lax
from jax.experimental import pallas as pl
from jax.experimental.pallas import tpu as pltpu
from jax.experimental.pallas import tpu_sc as plsc

_B = 16384
_F = 26
_V = 100000
_D = 32
_DENSE = 13

_NC = 2
_NS = 16
_NW = _NC * _NS

_NR = (_B * _F) // 128
_RPW = _NR // _NW


def _sc_gather(didx2, lidx2, dtab, ltab):
    mesh = plsc.VectorSubcoreMesh(
        core_axis_name="c", subcore_axis_name="s",
        num_cores=_NC, num_subcores=_NS)

    @functools.partial(
        pl.kernel,
        out_type=(
            jax.ShapeDtypeStruct((_NR, 128, _D), jnp.float32),
            jax.ShapeDtypeStruct((_NR, 128), jnp.float32),
        ),
        mesh=mesh,
        compiler_params=pltpu.CompilerParams(use_tc_tiling_on_sc=False),
        scratch_types=[
            pltpu.VMEM((_RPW, 128), jnp.int32),
            pltpu.VMEM((_RPW, 128), jnp.int32),
            pltpu.VMEM((_RPW, 128), jnp.float32),
            pltpu.VMEM((128, _D), jnp.float32),
            pltpu.VMEM((128, _D), jnp.float32),
            pltpu.SemaphoreType.DMA,
            pltpu.SemaphoreType.DMA,
            pltpu.SemaphoreType.DMA,
        ],
    )
    def body(didx_hbm, lidx_hbm, dtab_hbm, ltab_hbm, fm_out, lin_out,
             didx_v, lidx_v, lval_v, buf0, buf1, sem_a, sem_b, sem_l):
        wid = lax.axis_index("s") * _NC + lax.axis_index("c")
        row0 = wid * _RPW

        pltpu.sync_copy(didx_hbm.at[pl.ds(row0, _RPW)], didx_v)
        pltpu.sync_copy(lidx_hbm.at[pl.ds(row0, _RPW)], lidx_v)

        iota = lax.iota(jnp.int32, 16)

        def addoff(j, carry):
            for t in range(8):
                sl = pl.ds(t * 16, 16)
                off = (((j * 128 + t * 16) + iota) % _F) * _V
                didx_v[j, sl] = didx_v[j, sl] + off
                lidx_v[j, sl] = lidx_v[j, sl] + off
            return carry
        lax.fori_loop(0, _RPW, addoff, 0)

        def lin_group(g, carry):
            for t in range(8):
                r = g * 8 + t
                pltpu.async_copy(ltab_hbm.at[lidx_v.at[r]], lval_v.at[r], sem_l)
            pltpu.make_async_copy(
                lin_out.at[pl.ds(row0, 8)],
                lval_v.at[pl.ds(g * 8, 8)], sem_l).wait()
            return carry
        lax.fori_loop(0, 13, lin_group, 0)
        pltpu.sync_copy(lval_v, lin_out.at[pl.ds(row0, _RPW)])

        pltpu.async_copy(dtab_hbm.at[didx_v.at[0]], buf0, sem_a)

        def dnn_body(k, carry):
            j0 = 2 * k
            j1 = j0 + 1
            pltpu.async_copy(dtab_hbm.at[didx_v.at[j1]], buf1, sem_b)
            pltpu.make_async_copy(fm_out.at[0], buf0, sem_a).wait()
            pltpu.sync_copy(buf0, fm_out.at[row0 + j0])

            @pl.when(k < _RPW // 2 - 1)
            def _():
                pltpu.async_copy(dtab_hbm.at[didx_v.at[j0 + 2]], buf0, sem_a)

            pltpu.make_async_copy(fm_out.at[0], buf1, sem_b).wait()
            pltpu.sync_copy(buf1, fm_out.at[row0 + j1])
            return carry
        lax.fori_loop(0, _RPW // 2, dnn_body, 0)

    return body(didx2, lidx2, dtab, ltab)


def _tc_head(fm2, lval, dnn_dense, lin_dense,
             w1s, w1d, b1, g1, be1, w2, b2, g2, be2, w3, b3, lin_w, lin_b):
    bb = 512
    grid = (_B // bb,)

    def body(fm_ref, lv_ref, dd_ref, ld_ref,
             w1s_ref, w1d_ref, b1_ref, g1_ref, be1_ref,
             w2_ref, b2_ref, g2_ref, be2_ref,
             w3_ref, b3_ref, linw_ref, linb_ref, out_ref):
        fm = fm_ref[...]
        r = lax.broadcasted_iota(jnp.int32, (_F * _D, _D), 0)
        c = lax.broadcasted_iota(jnp.int32, (_F * _D, _D), 1)
        s = (r % _D == c).astype(jnp.float32)
        dn = (((1,), (1,)), ((), ()))
        mm = lambda x, w: lax.dot_general(
            x, w, dimension_numbers=dn, preferred_element_type=jnp.float32)
        sum_e = lax.dot_general(fm, s, dimension_numbers=(((1,), (0,)), ((), ())),
                                preferred_element_type=jnp.float32)
        ssq = lax.dot_general(fm * fm, s, dimension_numbers=(((1,), (0,)), ((), ())),
                              preferred_element_type=jnp.float32)
        cross = 0.5 * jnp.sum(sum_e * sum_e - ssq, axis=1, keepdims=True)

        h = mm(fm, w1s_ref[...]) + mm(dd_ref[...], w1d_ref[...]) + b1_ref[...]
        h = jnp.maximum(h, 0.0)
        m = jnp.mean(h, axis=1, keepdims=True)
        xc = h - m
        v = jnp.mean(xc * xc, axis=1, keepdims=True)
        h = xc * lax.rsqrt(v + 1e-5) * g1_ref[...] + be1_ref[...]

        h = jnp.maximum(mm(h, w2_ref[...]) + b2_ref[...], 0.0)
        m = jnp.mean(h, axis=1, keepdims=True)
        xc = h - m
        v = jnp.mean(xc * xc, axis=1, keepdims=True)
        h = xc * lax.rsqrt(v + 1e-5) * g2_ref[...] + be2_ref[...]

        dnn_logit = jnp.maximum(
            jnp.sum(h * w3_ref[...], axis=1, keepdims=True) + b3_ref[0, 0], 0.0)

        lin_logit = (jnp.sum(ld_ref[...] * linw_ref[...], axis=1, keepdims=True)
                     + linb_ref[0, 0]
                     + jnp.sum(lv_ref[...], axis=1, keepdims=True))
        out_ref[...] = lin_logit + dnn_logit + cross

    full = lambda shape: pl.BlockSpec(shape, lambda i: (0, 0))
    return pl.pallas_call(
        body,
        grid=grid,
        in_specs=[
            pl.BlockSpec((bb, _F * _D), lambda i: (i, 0)),
            pl.BlockSpec((bb, _F), lambda i: (i, 0)),
            pl.BlockSpec((bb, _DENSE), lambda i: (i, 0)),
            pl.BlockSpec((bb, _DENSE), lambda i: (i, 0)),
            full((128, _F * _D)), full((128, _DENSE)),
            full((1, 128)), full((1, 128)), full((1, 128)),
            full((64, 128)), full((1, 64)), full((1, 64)), full((1, 64)),
            full((1, 64)), full((1, 1)), full((1, _DENSE)), full((1, 1)),
        ],
        out_specs=pl.BlockSpec((bb, 1), lambda i: (i, 0)),
        out_shape=jax.ShapeDtypeStruct((_B, 1), jnp.float32),
    )(fm2, lval, dnn_dense, lin_dense,
      w1s, w1d, b1, g1, be1, w2, b2, g2, be2, w3, b3, lin_w, lin_b)


def kernel(linear_dense_data, dnn_dense_data, linear_tables, dnn_tables,
           lin_W, lin_b, W1, b1, ln1_g, ln1_b, W2, b2, ln2_g, ln2_b, W3, b3,
           linear_sparse_data, dnn_sparse_data):
    didx2 = dnn_sparse_data.astype(jnp.int32).reshape(_NR, 128)
    lidx2 = linear_sparse_data.astype(jnp.int32).reshape(_NR, 128)
    dtab = dnn_tables.reshape(_F * _V, _D)
    ltab = linear_tables.reshape(_F * _V)

    fm_rows, lin_rows = _sc_gather(didx2, lidx2, dtab, ltab)
    fm2 = fm_rows.reshape(_B, _F * _D)
    lval = lin_rows.reshape(_B, _F)

    w1d = W1[:, :_DENSE]
    w1s = W1[:, _DENSE:]
    out = _tc_head(
        fm2, lval, dnn_dense_data, linear_dense_data,
        w1s, w1d,
        b1.reshape(1, 128), ln1_g.reshape(1, 128), ln1_b.reshape(1, 128),
        W2, b2.reshape(1, 64), ln2_g.reshape(1, 64), ln2_b.reshape(1, 64),
        W3.reshape(1, 64), b3.reshape(1, 1),
        lin_W.reshape(1, _DENSE), lin_b.reshape(1, 1))
    return out
